# Initial kernel scaffold; baseline (speedup 1.0000x reference)
import jax
import jax.numpy as jnp
from jax.experimental import pallas as pl


def kernel(x, edge_index, W1, b1, gamma, beta, W2, b2):
    raise NotImplementedError("write your pallas kernel here")


# speedup vs baseline: 17.2197x; 17.2197x over previous
import functools

import jax
import jax.numpy as jnp
from jax import lax
from jax.experimental import pallas as pl
from jax.experimental.pallas import tpu as pltpu
from jax.experimental.pallas import tpu_sc as plsc

N_NODES = 10000
IN_DIM = 128
HID = 64
OUT_DIM = 128

NC = 2
NS = 16
L = 16
NW = NC * NS
CHUNK = 128
ROWS_PER_TILE = 640
NROWS = NS * ROWS_PER_TILE
DUMMY = N_NODES
DEGW = 16


def _mesh():
    return plsc.VectorSubcoreMesh(core_axis_name="c", subcore_axis_name="s")


@functools.lru_cache(maxsize=None)
def _deg_call(k_chunks):
    @functools.partial(
        pl.kernel,
        out_type=jax.ShapeDtypeStruct((NC, NROWS, DEGW), jnp.float32),
        mesh=_mesh(),
        scratch_types=[
            pltpu.VMEM((k_chunks, CHUNK), jnp.int32),
            pltpu.VMEM((CHUNK, DEGW), jnp.float32),
            pltpu.VMEM((CHUNK, DEGW), jnp.float32),
            pltpu.VMEM_SHARED((NROWS, DEGW), jnp.float32),
        ],
    )
    def deg_kernel(cols_hbm, out_hbm, colbuf, zbuf, obuf, hist):
        cid = lax.axis_index("c")
        sid = lax.axis_index("s")
        wid = sid * NC + cid

        def fill(i, _):
            zbuf[i, :] = jnp.zeros((L,), jnp.float32)
            obuf[i, :] = jnp.full((L,), 1.0, jnp.float32)
            return 0

        lax.fori_loop(0, CHUNK, fill, 0)

        for b in range(ROWS_PER_TILE // CHUNK):
            pltpu.sync_copy(zbuf, hist.at[pl.ds(sid * ROWS_PER_TILE + b * CHUNK, CHUNK)])
        plsc.subcore_barrier()

        pltpu.sync_copy(cols_hbm.at[wid], colbuf)

        def step(j, _):
            pltpu.sync_copy(obuf, hist.at[colbuf.at[j]], add=True)
            return 0

        lax.fori_loop(0, k_chunks, step, 0)
        plsc.subcore_barrier()

        for b in range(ROWS_PER_TILE // CHUNK):
            r0 = sid * ROWS_PER_TILE + b * CHUNK
            pltpu.sync_copy(hist.at[pl.ds(r0, CHUNK)], zbuf)
            pltpu.sync_copy(zbuf, out_hbm.at[cid, pl.ds(r0, CHUNK)])

    return deg_kernel


@functools.lru_cache(maxsize=None)
def _accum_call(d, k_chunks):
    @functools.partial(
        pl.kernel,
        out_type=jax.ShapeDtypeStruct((NC, NROWS, d), jnp.float32),
        mesh=_mesh(),
        scratch_types=[
            pltpu.VMEM((k_chunks, CHUNK), jnp.int32),
            pltpu.VMEM((k_chunks, CHUNK), jnp.int32),
            pltpu.VMEM((CHUNK, d), jnp.float32),
            pltpu.SemaphoreType.DMA,
            pltpu.VMEM_SHARED((NROWS, d), jnp.float32),
        ],
        compiler_params=pltpu.CompilerParams(use_tc_tiling_on_sc=False),
    )
    def accum_kernel(g_hbm, rows_hbm, cols_hbm, out_hbm, rowbuf, colbuf, buf0, sem, accum):
        cid = lax.axis_index("c")
        sid = lax.axis_index("s")
        wid = sid * NC + cid

        def fill(i, _):
            for k in range(d // L):
                buf0[i, pl.ds(k * L, L)] = jnp.zeros((L,), jnp.float32)
            return 0

        lax.fori_loop(0, CHUNK, fill, 0)

        for b in range(ROWS_PER_TILE // CHUNK):
            pltpu.sync_copy(buf0, accum.at[pl.ds(sid * ROWS_PER_TILE + b * CHUNK, CHUNK)])
        plsc.subcore_barrier()

        pltpu.sync_copy(rows_hbm.at[wid], rowbuf)
        pltpu.sync_copy(cols_hbm.at[wid], colbuf)

        def step(j, _):
            pltpu.async_copy(g_hbm.at[rowbuf.at[j]], buf0, sem).wait()
            pltpu.sync_copy(buf0, accum.at[colbuf.at[j]], add=True)
            return 0

        lax.fori_loop(0, k_chunks, step, 0)
        plsc.subcore_barrier()

        for b in range(ROWS_PER_TILE // CHUNK):
            r0 = sid * ROWS_PER_TILE + b * CHUNK
            pltpu.sync_copy(accum.at[pl.ds(r0, CHUNK)], buf0)
            pltpu.sync_copy(buf0, out_hbm.at[cid, pl.ds(r0, CHUNK)])

    return accum_kernel


def _dinv_from_hist(hist):
    deg = hist[0, :N_NODES, 0] + hist[1, :N_NODES, 0] + 1.0
    return lax.rsqrt(deg)


def _t1_body(hist_ref, x_ref, w1_ref, g1_ref):
    dinv = _dinv_from_hist(hist_ref[...])
    h = jnp.dot(x_ref[...], w1_ref[...].T, preferred_element_type=jnp.float32)
    g1_ref[...] = h * dinv[:, None]


def _t2_body(hist_ref, p_ref, g1_ref, b1_ref, gamma_ref, beta_ref, w2_ref, g2_ref):
    dinv = _dinv_from_hist(hist_ref[...])
    p = p_ref[...]
    a = (p[0, :N_NODES] + p[1, :N_NODES] + g1_ref[...]) * dinv[:, None] + b1_ref[...]
    mean = jnp.mean(a, axis=0)
    var = jnp.mean((a - mean) ** 2, axis=0)
    bn = (a - mean) * lax.rsqrt(var + 1e-5) * gamma_ref[...] + beta_ref[...]
    r = jnp.maximum(bn, 0.0)
    h2 = jnp.dot(r, w2_ref[...].T, preferred_element_type=jnp.float32)
    g2_ref[...] = h2 * dinv[:, None]


def _t3_body(hist_ref, q_ref, g2_ref, b2_ref, out_ref):
    dinv = _dinv_from_hist(hist_ref[...])
    q = q_ref[...]
    out_ref[...] = (q[0, :N_NODES] + q[1, :N_NODES] + g2_ref[...]) * dinv[:, None] + b2_ref[...]


def kernel(x, edge_index, W1, b1, gamma, beta, W2, b2):
    e = edge_index.shape[1]
    k_chunks = -(-e // (NW * CHUNK))
    e_pad = k_chunks * NW * CHUNK
    pad = e_pad - e
    rows = jnp.concatenate(
        [edge_index[0], jnp.zeros((pad,), jnp.int32)]).reshape(NW, k_chunks, CHUNK)
    cols = jnp.concatenate(
        [edge_index[1], jnp.full((pad,), DUMMY, jnp.int32)]).reshape(NW, k_chunks, CHUNK)

    hist = _deg_call(k_chunks)(cols)

    g1 = pl.pallas_call(
        _t1_body,
        out_shape=jax.ShapeDtypeStruct((N_NODES, HID), jnp.float32),
    )(hist, x, W1)

    p1 = _accum_call(HID, k_chunks)(g1, rows, cols)

    g2 = pl.pallas_call(
        _t2_body,
        out_shape=jax.ShapeDtypeStruct((N_NODES, OUT_DIM), jnp.float32),
    )(hist, p1, g1, b1.reshape(1, HID), gamma.reshape(1, HID),
      beta.reshape(1, HID), W2)

    p2 = _accum_call(OUT_DIM, k_chunks)(g2, rows, cols)

    out = pl.pallas_call(
        _t3_body,
        out_shape=jax.ShapeDtypeStruct((N_NODES, OUT_DIM), jnp.float32),
    )(hist, p2, g2, b2.reshape(1, OUT_DIM))

    return out
